# baseline (speedup 1.0000x reference)
import jax
import jax.numpy as jnp
from jax.experimental import pallas as pl
from jax.experimental.pallas import tpu as pltpu

_R = 3
_M = 64
_N = 32768


def _gather_body(mask_ref, x_ref, o_ref, sem):
    copies = []
    for i in range(_R):
        count = 0
        src = 0
        for row in range(_R):
            keep = 1 - mask_ref[row]
            hit = jnp.logical_and(count == i, keep == 1)
            src = jnp.where(hit, row, src)
            count = count + keep
        copies.append(
            pltpu.make_async_copy(x_ref.at[src], o_ref.at[i], sem.at[i])
        )
    for c in copies:
        c.start()
    for c in copies:
        c.wait()


def kernel(x, bool_tensor):
    mask_i32 = bool_tensor.astype(jnp.int32)
    out = pl.pallas_call(
        _gather_body,
        grid_spec=pltpu.PrefetchScalarGridSpec(
            num_scalar_prefetch=1,
            grid=(),
            in_specs=[pl.BlockSpec(memory_space=pl.ANY)],
            out_specs=pl.BlockSpec(memory_space=pl.ANY),
            scratch_shapes=[pltpu.SemaphoreType.DMA((_R,))],
        ),
        out_shape=jax.ShapeDtypeStruct((_R, _M, _N), x.dtype),
    )(mask_i32, x)
    return out


# speedup vs baseline: 43.2893x; 43.2893x over previous
import jax
import jax.numpy as jnp
from jax.experimental import pallas as pl
from jax.experimental.pallas import tpu as pltpu

_R = 3
_M = 64
_N = 32768


def _copy_body(mask_ref, x_ref, o_ref):
    o_ref[...] = x_ref[...]


def _src_index_map(i, mask_ref):
    count = 0
    src = 0
    for row in range(_R):
        keep = 1 - mask_ref[row]
        hit = jnp.logical_and(count == i, keep == 1)
        src = jnp.where(hit, row, src)
        count = count + keep
    return (src, 0, 0)


def kernel(x, bool_tensor):
    mask_i32 = bool_tensor.astype(jnp.int32)
    out = pl.pallas_call(
        _copy_body,
        grid_spec=pltpu.PrefetchScalarGridSpec(
            num_scalar_prefetch=1,
            grid=(_R,),
            in_specs=[
                pl.BlockSpec((1, _M, _N), _src_index_map),
            ],
            out_specs=pl.BlockSpec((1, _M, _N), lambda i, m: (i, 0, 0)),
        ),
        out_shape=jax.ShapeDtypeStruct((_R, _M, _N), x.dtype),
    )(mask_i32, x)
    return out
